# Initial kernel scaffold; baseline (speedup 1.0000x reference)
import jax
import jax.numpy as jnp
from jax.experimental import pallas as pl


def kernel(box_preds, class_logits, anchors):
    raise NotImplementedError("write your pallas kernel here")


# speedup vs baseline: 4.5820x; 4.5820x over previous
import jax
import jax.numpy as jnp
from jax.experimental import pallas as pl
from jax.experimental.pallas import tpu as pltpu

N = 20000
C = 80
K = 300
IOU_THR = 0.5
BBOX_XFORM_CLIP = 4.135166556742356

NPAD = 157 * 128


def _kernel(box_ref, logit_ref, anc_ref,
            boxes_out_ref, scores_out_ref, classes_out_ref,
            s_ref, rmax_ref, selb_ref, sela_ref, iou_ref):
    s = jax.nn.sigmoid(logit_ref[...])
    s_ref[...] = s
    rm = jnp.max(s, axis=1)
    rm = jnp.concatenate([rm, jnp.full((NPAD - N,), -1.0, jnp.float32)])
    rmax_ref[...] = rm.reshape(157, 128)

    p_io = jax.lax.broadcasted_iota(jnp.int32, (157, 128), 0)
    q_io = jax.lax.broadcasted_iota(jnp.int32, (157, 128), 1)
    ridx = p_io * 128 + q_io
    lane_io = jax.lax.broadcasted_iota(jnp.int32, (1, C), 1)
    BIG = jnp.int32(1 << 30)

    kiota = jax.lax.broadcasted_iota(jnp.int32, (K, 1), 0)

    def body(k, carry):
        rmax = rmax_ref[...]
        v = jnp.max(rmax)
        r = jnp.min(jnp.where(rmax == v, ridx, BIG))
        row = s_ref[pl.ds(r, 1), :]
        lane = jnp.min(jnp.where(row == v, lane_io, BIG))
        new_row = jnp.where(lane_io == lane, -1.0, row)
        s_ref[pl.ds(r, 1), :] = new_row
        m2 = jnp.max(new_row)
        rmax_ref[...] = jnp.where(ridx == r, m2, rmax)
        at_k = kiota == k
        scores_out_ref[...] = jnp.where(at_k, v, scores_out_ref[...])
        classes_out_ref[...] = jnp.where(at_k, lane, classes_out_ref[...])
        selb_ref[...] = jnp.where(at_k, box_ref[pl.ds(r, 1), :], selb_ref[...])
        sela_ref[...] = jnp.where(at_k, anc_ref[pl.ds(r, 1), :], sela_ref[...])
        return carry

    jax.lax.fori_loop(0, K, body, jnp.int32(0))

    rel = selb_ref[...]
    an = sela_ref[...]
    wa = an[:, 2:3] - an[:, 0:1]
    ha = an[:, 3:4] - an[:, 1:2]
    cxa = an[:, 0:1] + 0.5 * wa
    cya = an[:, 1:2] + 0.5 * ha
    dx = rel[:, 0:1] / 10.0
    dy = rel[:, 1:2] / 10.0
    dw = jnp.minimum(rel[:, 2:3] / 5.0, BBOX_XFORM_CLIP)
    dh = jnp.minimum(rel[:, 3:4] / 5.0, BBOX_XFORM_CLIP)
    cx = dx * wa + cxa
    cy = dy * ha + cya
    w = jnp.exp(dw) * wa
    h = jnp.exp(dh) * ha
    x1 = cx - 0.5 * w
    y1 = cy - 0.5 * h
    x2 = cx + 0.5 * w
    y2 = cy + 0.5 * h
    dec = jnp.concatenate([x1, y1, x2, y2], axis=1)

    area = jnp.clip(x2 - x1, 0.0) * jnp.clip(y2 - y1, 0.0)
    x1t, y1t = jnp.transpose(x1), jnp.transpose(y1)
    x2t, y2t = jnp.transpose(x2), jnp.transpose(y2)
    iw = jnp.clip(jnp.minimum(x2, x2t) - jnp.maximum(x1, x1t), 0.0)
    ih = jnp.clip(jnp.minimum(y2, y2t) - jnp.maximum(y1, y1t), 0.0)
    inter = iw * ih
    union = area + jnp.transpose(area) - inter
    iou = jnp.where(union > 0.0, inter / union, 0.0)

    iou_ref[...] = iou
    ar = jax.lax.broadcasted_iota(jnp.int32, (1, K), 1)

    def nms_body(i, keep):
        row_i = iou_ref[pl.ds(i, 1), :]
        k_i = jnp.max(jnp.where(ar == i, keep, 0.0))
        sup = (row_i > IOU_THR) & (ar > i) & (k_i > 0.5)
        return keep * jnp.where(sup, 0.0, 1.0)

    keep = jax.lax.fori_loop(0, K, nms_body, jnp.ones((1, K), jnp.float32))
    keep_col = jnp.transpose(keep) > 0.5

    boxes_out_ref[...] = jnp.where(keep_col, dec, 0.0)
    scores_out_ref[...] = jnp.where(keep_col, scores_out_ref[...], 0.0)
    classes_out_ref[...] = jnp.where(keep_col, classes_out_ref[...], -1)


def kernel(box_preds, class_logits, anchors):
    boxes, scores, classes = pl.pallas_call(
        _kernel,
        out_shape=(
            jax.ShapeDtypeStruct((K, 4), jnp.float32),
            jax.ShapeDtypeStruct((K, 1), jnp.float32),
            jax.ShapeDtypeStruct((K, 1), jnp.int32),
        ),
        scratch_shapes=[
            pltpu.VMEM((N, C), jnp.float32),
            pltpu.VMEM((157, 128), jnp.float32),
            pltpu.VMEM((K, 4), jnp.float32),
            pltpu.VMEM((K, 4), jnp.float32),
            pltpu.VMEM((K, K), jnp.float32),
        ],
    )(box_preds[0], class_logits[0], anchors)
    return boxes, scores[:, 0], classes[:, 0]


# speedup vs baseline: 4.6931x; 1.0242x over previous
import jax
import jax.numpy as jnp
from jax.experimental import pallas as pl
from jax.experimental.pallas import tpu as pltpu

N = 20000
C = 80
K = 300
IOU_THR = 0.5
BBOX_XFORM_CLIP = 4.135166556742356

NPAD = 157 * 128


def _kernel(box_ref, logit_ref, anc_ref,
            boxes_out_ref, scores_out_ref, classes_out_ref,
            s_ref, rmax_ref, selb_ref, sela_ref, iou_ref):
    s = jax.nn.sigmoid(logit_ref[...])
    s_ref[...] = s
    rm = jnp.max(s, axis=1)
    rm = jnp.concatenate([rm, jnp.full((NPAD - N,), -1.0, jnp.float32)])
    rmax_ref[...] = rm.reshape(157, 128)

    p_io = jax.lax.broadcasted_iota(jnp.int32, (157, 128), 0)
    q_io = jax.lax.broadcasted_iota(jnp.int32, (157, 128), 1)
    ridx = p_io * 128 + q_io
    lane_io = jax.lax.broadcasted_iota(jnp.int32, (1, C), 1)
    BIG = jnp.int32(1 << 30)

    q_row = jax.lax.broadcasted_iota(jnp.int32, (1, 128), 1)

    def body(k, carry):
        rmax = rmax_ref[...]
        v = jnp.max(rmax)
        r = jnp.min(jnp.where(rmax == v, ridx, BIG))
        row = s_ref[pl.ds(r, 1), :]
        lane = jnp.min(jnp.where(row == v, lane_io, BIG))
        new_row = jnp.where(lane_io == lane, -1.0, row)
        s_ref[pl.ds(r, 1), :] = new_row
        m2 = jnp.max(new_row)
        p = r // 128
        rrow = rmax_ref[pl.ds(p, 1), :]
        rmax_ref[pl.ds(p, 1), :] = jnp.where(q_row == r % 128, m2, rrow)
        scores_out_ref[pl.ds(k, 1), :] = v.reshape(1, 1)
        classes_out_ref[pl.ds(k, 1), :] = lane.reshape(1, 1)
        selb_ref[pl.ds(k, 1), :] = box_ref[pl.ds(r, 1), :]
        sela_ref[pl.ds(k, 1), :] = anc_ref[pl.ds(r, 1), :]
        return carry

    jax.lax.fori_loop(0, K, body, jnp.int32(0))

    rel = selb_ref[...]
    an = sela_ref[...]
    wa = an[:, 2:3] - an[:, 0:1]
    ha = an[:, 3:4] - an[:, 1:2]
    cxa = an[:, 0:1] + 0.5 * wa
    cya = an[:, 1:2] + 0.5 * ha
    dx = rel[:, 0:1] / 10.0
    dy = rel[:, 1:2] / 10.0
    dw = jnp.minimum(rel[:, 2:3] / 5.0, BBOX_XFORM_CLIP)
    dh = jnp.minimum(rel[:, 3:4] / 5.0, BBOX_XFORM_CLIP)
    cx = dx * wa + cxa
    cy = dy * ha + cya
    w = jnp.exp(dw) * wa
    h = jnp.exp(dh) * ha
    x1 = cx - 0.5 * w
    y1 = cy - 0.5 * h
    x2 = cx + 0.5 * w
    y2 = cy + 0.5 * h
    dec = jnp.concatenate([x1, y1, x2, y2], axis=1)

    area = jnp.clip(x2 - x1, 0.0) * jnp.clip(y2 - y1, 0.0)
    x1t, y1t = jnp.transpose(x1), jnp.transpose(y1)
    x2t, y2t = jnp.transpose(x2), jnp.transpose(y2)
    iw = jnp.clip(jnp.minimum(x2, x2t) - jnp.maximum(x1, x1t), 0.0)
    ih = jnp.clip(jnp.minimum(y2, y2t) - jnp.maximum(y1, y1t), 0.0)
    inter = iw * ih
    union = area + jnp.transpose(area) - inter
    iou = jnp.where(union > 0.0, inter / union, 0.0)

    iou_ref[...] = iou
    ar = jax.lax.broadcasted_iota(jnp.int32, (1, K), 1)

    def nms_body(i, keep):
        row_i = iou_ref[pl.ds(i, 1), :]
        k_i = jnp.max(jnp.where(ar == i, keep, 0.0))
        sup = (row_i > IOU_THR) & (ar > i) & (k_i > 0.5)
        return keep * jnp.where(sup, 0.0, 1.0)

    keep = jax.lax.fori_loop(0, K, nms_body, jnp.ones((1, K), jnp.float32))
    keep_col = jnp.transpose(keep) > 0.5

    boxes_out_ref[...] = jnp.where(keep_col, dec, 0.0)
    scores_out_ref[...] = jnp.where(keep_col, scores_out_ref[...], 0.0)
    classes_out_ref[...] = jnp.where(keep_col, classes_out_ref[...], -1)


def kernel(box_preds, class_logits, anchors):
    boxes, scores, classes = pl.pallas_call(
        _kernel,
        out_shape=(
            jax.ShapeDtypeStruct((K, 4), jnp.float32),
            jax.ShapeDtypeStruct((K, 1), jnp.float32),
            jax.ShapeDtypeStruct((K, 1), jnp.int32),
        ),
        scratch_shapes=[
            pltpu.VMEM((N, C), jnp.float32),
            pltpu.VMEM((157, 128), jnp.float32),
            pltpu.VMEM((K, 4), jnp.float32),
            pltpu.VMEM((K, 4), jnp.float32),
            pltpu.VMEM((K, K), jnp.float32),
        ],
    )(box_preds[0], class_logits[0], anchors)
    return boxes, scores[:, 0], classes[:, 0]
